# Initial kernel scaffold; baseline (speedup 1.0000x reference)
import jax
import jax.numpy as jnp
from jax.experimental import pallas as pl


def kernel(X, mask, R_idx, chain_labels, membrane_per_residue_labels, pos_W, pos_b, edge_W, ln_e_g, ln_e_b, node_W, ln_n_g, ln_n_b):
    raise NotImplementedError("write your pallas kernel here")


# speedup vs baseline: 2.6100x; 2.6100x over previous
import functools

import jax
import jax.numpy as jnp
from jax import lax
from jax.experimental import pallas as pl
from jax.experimental.pallas import tpu as pltpu
from jax.experimental.pallas import tpu_sc as plsc

_K = 48
_NUM_RBF = 16
_MAXREL = 32

_N, _CA, _C, _O, _CB = 0, 3, 6, 9, 12
_PAIRS = (
    (_CA, _CA),
    (_N, _N), (_C, _C), (_O, _O), (_CB, _CB),
    (_CA, _N), (_CA, _C), (_CA, _O), (_CA, _CB),
    (_N, _C), (_N, _O), (_N, _CB), (_CB, _C), (_CB, _O), (_O, _C),
    (_N, _CA), (_C, _CA), (_O, _CA), (_CB, _CA),
    (_C, _N), (_O, _N), (_CB, _N), (_C, _CB), (_O, _CB), (_C, _O),
)


def _stage1_body(x_ref, cat_ref, meta_ref, eidx_ref, coords_ref, dcur):
    x = x_ref[...]
    n = x[:, 0:3]
    ca = x[:, 3:6]
    cc = x[:, 6:9]
    b = ca - n
    c = cc - ca
    ax = b[:, 1:2] * c[:, 2:3] - b[:, 2:3] * c[:, 1:2]
    ay = b[:, 2:3] * c[:, 0:1] - b[:, 0:1] * c[:, 2:3]
    az = b[:, 0:1] * c[:, 1:2] - b[:, 1:2] * c[:, 0:1]
    a = jnp.concatenate([ax, ay, az], axis=1)
    cb = -0.58273431 * a + 0.56802827 * b - 0.54067466 * c + ca
    coords_ref[:, 0:12] = x
    coords_ref[:, 12:15] = cb
    coords_ref[:, 15:16] = meta_ref[...]

    rb, l = dcur.shape
    dx = ca[:, 0:1] - cat_ref[0:1, :]
    dy = ca[:, 1:2] - cat_ref[1:2, :]
    dz = ca[:, 2:3] - cat_ref[2:3, :]
    d = jnp.sqrt(dx * dx + dy * dy + dz * dz + 1e-6)
    iota = lax.broadcasted_iota(jnp.int32, (rb, l), 1)
    dcur[...] = d
    for t in range(_K):
        d = dcur[...]
        m = jnp.min(d, axis=1, keepdims=True)
        idx = jnp.min(jnp.where(d == m, iota, l), axis=1, keepdims=True)
        eidx_ref[:, t:t + 1] = idx
        dcur[...] = jnp.where(iota == idx, jnp.inf, d)


def _stage1(x2, cat, meta, interpret=False):
    l = x2.shape[0]
    rb = 128
    grid = (l // rb,)
    return pl.pallas_call(
        _stage1_body,
        grid=grid,
        in_specs=[
            pl.BlockSpec((rb, 12), lambda i: (i, 0)),
            pl.BlockSpec((3, l), lambda i: (0, 0)),
            pl.BlockSpec((rb, 1), lambda i: (i, 0)),
        ],
        out_specs=[
            pl.BlockSpec((rb, _K), lambda i: (i, 0)),
            pl.BlockSpec((rb, 16), lambda i: (i, 0)),
        ],
        out_shape=[
            jax.ShapeDtypeStruct((l, _K), jnp.int32),
            jax.ShapeDtypeStruct((l, 16), jnp.float32),
        ],
        scratch_shapes=[pltpu.VMEM((rb, l), jnp.float32)],
        interpret=interpret,
    )(x2, cat, meta)


def _stage2_gather(coords16, eidx_flat):
    ne = eidx_flat.shape[0]
    nw = 32
    bpw = ne // nw
    mesh = plsc.VectorSubcoreMesh(core_axis_name="c", subcore_axis_name="s")

    @functools.partial(
        pl.kernel,
        mesh=mesh,
        out_type=jax.ShapeDtypeStruct((ne, 16), jnp.float32),
        compiler_params=pltpu.CompilerParams(use_tc_tiling_on_sc=False),
        scratch_types=[
            pltpu.VMEM((bpw,), jnp.int32),
            pltpu.VMEM((bpw, 16), jnp.float32),
            pltpu.SemaphoreType.DMA,
        ],
    )
    def k(table_hbm, idx_hbm, out_hbm, idx_v, rows_v, sem):
        wid = lax.axis_index("s") * 2 + lax.axis_index("c")
        base = wid * bpw
        pltpu.sync_copy(idx_hbm.at[pl.ds(base, bpw)], idx_v)
        pltpu.async_copy(table_hbm.at[idx_v], rows_v, sem).wait()
        pltpu.sync_copy(rows_v, out_hbm.at[pl.ds(base, bpw)])

    return k(coords16, eidx_flat)


def _stage3_body(yt_ref, at_ref, mem_ref, mus_ref, wpc_ref, wrbf_ref,
                 bias_ref, g_ref, b_ref, nodew_ref, ng_ref, nb_ref,
                 e_ref, v_ref, acc_ref):
    yt = yt_ref[...]
    at = at_ref[...]

    m_a = at[15:16, :]
    m_b = yt[15:16, :]
    r_a = jnp.floor(m_a * 0.25)
    r_b = jnp.floor(m_b * 0.25)
    ch_eq = (m_a - 4.0 * r_a) == (m_b - 4.0 * r_b)
    off = r_a - r_b
    dpos = jnp.where(ch_eq, jnp.clip(off + _MAXREL, 0.0, 2.0 * _MAXREL),
                     2.0 * _MAXREL + 1.0)
    io66 = lax.broadcasted_iota(
        jnp.int32, (2 * _MAXREL + 2, 1), 0).astype(jnp.float32)
    onehot = (dpos == io66).astype(jnp.float32)
    dn = (((0,), (0,)), ((), ()))
    acc_ref[...] = lax.dot_general(
        onehot, wpc_ref[...], dn,
        preferred_element_type=jnp.float32) + bias_ref[...]

    musc = mus_ref[...]
    inv_sigma = _NUM_RBF / (22.0 - 2.0)
    for p, (ao, bo) in enumerate(_PAIRS):
        dx = at[ao:ao + 1, :] - yt[bo:bo + 1, :]
        dy = at[ao + 1:ao + 2, :] - yt[bo + 1:bo + 2, :]
        dz = at[ao + 2:ao + 3, :] - yt[bo + 2:bo + 3, :]
        dist = jnp.sqrt(dx * dx + dy * dy + dz * dz + 1e-6)
        z = (dist - musc) * inv_sigma
        rbf = jnp.exp(-(z * z))
        acc_ref[...] = acc_ref[...] + lax.dot_general(
            rbf, wrbf_ref[_NUM_RBF * p:_NUM_RBF * (p + 1), :], dn,
            preferred_element_type=jnp.float32)

    acc = acc_ref[...]
    mu = jnp.mean(acc, axis=-1, keepdims=True)
    xc = acc - mu
    var = jnp.mean(xc * xc, axis=-1, keepdims=True)
    e_ref[...] = xc / jnp.sqrt(var + 1e-5) * g_ref[...] + b_ref[...]

    mem = mem_ref[...]
    io3 = lax.broadcasted_iota(jnp.int32, (mem.shape[0], 3), 1)
    oh3 = (mem == io3).astype(jnp.float32)
    v = jnp.dot(oh3, nodew_ref[...], preferred_element_type=jnp.float32)
    vmu = jnp.mean(v, axis=-1, keepdims=True)
    vc = v - vmu
    vvar = jnp.mean(vc * vc, axis=-1, keepdims=True)
    v_ref[...] = vc / jnp.sqrt(vvar + 1e-5) * ng_ref[...] + nb_ref[...]


def _stage3(yt, at, mem2, mus_col, wpc, wrbf, bias_c, ln_e_g, ln_e_b,
            node_w, ln_n_g, ln_n_b, interpret=False):
    ne = yt.shape[1]
    l = mem2.shape[0]
    rb = 128
    eb = rb * _K
    grid = (l // rb,)
    full = lambda shape: pl.BlockSpec(shape, lambda i: tuple(0 for _ in shape))
    return pl.pallas_call(
        _stage3_body,
        grid=grid,
        in_specs=[
            pl.BlockSpec((16, eb), lambda i: (0, i)),
            pl.BlockSpec((16, eb), lambda i: (0, i)),
            pl.BlockSpec((rb, 1), lambda i: (i, 0)),
            full((_NUM_RBF, 1)),
            full((2 * _MAXREL + 2, 128)),
            full((_NUM_RBF * len(_PAIRS), 128)),
            full((1, 128)),
            full((1, 128)),
            full((1, 128)),
            full((3, 128)),
            full((1, 128)),
            full((1, 128)),
        ],
        out_specs=[
            pl.BlockSpec((eb, 128), lambda i: (i, 0)),
            pl.BlockSpec((rb, 128), lambda i: (i, 0)),
        ],
        out_shape=[
            jax.ShapeDtypeStruct((ne, 128), jnp.float32),
            jax.ShapeDtypeStruct((l, 128), jnp.float32),
        ],
        scratch_shapes=[pltpu.VMEM((eb, 128), jnp.float32)],
        interpret=interpret,
    )(yt, at, mem2, mus_col, wpc, wrbf, bias_c, ln_e_g, ln_e_b,
      node_w, ln_n_g, ln_n_b)


def kernel(X, mask, R_idx, chain_labels, membrane_per_residue_labels,
           pos_W, pos_b, edge_W, ln_e_g, ln_e_b, node_W, ln_n_g, ln_n_b):
    b, l = X.shape[0], X.shape[1]
    x2 = X.reshape(l, 12)
    cat = x2[:, 3:6].T
    meta = (4 * R_idx[0] + chain_labels[0]).astype(jnp.float32).reshape(l, 1)

    e_idx, coords16 = _stage1(x2, cat, meta)

    y = _stage2_gather(coords16, e_idx.reshape(l * _K))
    yt = y.T
    at = jnp.repeat(coords16.T, _K, axis=1)

    wpc = pos_W @ edge_W[:16]
    bias_c = (pos_b @ edge_W[:16]).reshape(1, 128)
    wrbf = edge_W[16:]
    mus_col = jnp.linspace(2.0, 22.0, _NUM_RBF).astype(jnp.float32).reshape(
        _NUM_RBF, 1)
    mem2 = membrane_per_residue_labels[0].astype(jnp.int32).reshape(l, 1)

    e, v = _stage3(yt, at, mem2, mus_col, wpc, wrbf, bias_c,
                   ln_e_g.reshape(1, 128), ln_e_b.reshape(1, 128),
                   node_W, ln_n_g.reshape(1, 128), ln_n_b.reshape(1, 128))

    return (v.reshape(b, l, 128), e.reshape(b, l, _K, 128),
            e_idx.reshape(b, l, _K))


def _run_cpu_stages(d, interpret=True):
    X, R_idx, chain = d["X"], d["R_idx"], d["chain_labels"]
    l = X.shape[1]
    x2 = X.reshape(l, 12)
    cat = x2[:, 3:6].T
    meta = (4 * R_idx[0] + chain[0]).astype(jnp.float32).reshape(l, 1)
    e_idx, coords16 = _stage1(x2, cat, meta, interpret=interpret)
    y = jnp.take_along_axis(coords16, e_idx.reshape(l * _K, 1), axis=0)
    yt = y.T
    at = jnp.repeat(coords16.T, _K, axis=1)
    wpc = d["pos_W"] @ d["edge_W"][:16]
    bias_c = (d["pos_b"] @ d["edge_W"][:16]).reshape(1, 128)
    wrbf = d["edge_W"][16:]
    mus_col = jnp.linspace(2.0, 22.0, _NUM_RBF).astype(jnp.float32).reshape(
        _NUM_RBF, 1)
    mem2 = d["membrane_per_residue_labels"][0].astype(jnp.int32).reshape(l, 1)
    e, v = _stage3(yt, at, mem2, mus_col, wpc, wrbf, bias_c,
                   d["ln_e_g"].reshape(1, 128), d["ln_e_b"].reshape(1, 128),
                   d["node_W"], d["ln_n_g"].reshape(1, 128),
                   d["ln_n_b"].reshape(1, 128), interpret=interpret)
    return e_idx, e.reshape(l, _K, 128), v


# speedup vs baseline: 3.7756x; 1.4466x over previous
import functools

import jax
import jax.numpy as jnp
from jax import lax
from jax.experimental import pallas as pl
from jax.experimental.pallas import tpu as pltpu
from jax.experimental.pallas import tpu_sc as plsc

_K = 48
_NUM_RBF = 16
_MAXREL = 32

_N, _CA, _C, _O, _CB = 0, 3, 6, 9, 12
_PAIRS = (
    (_CA, _CA),
    (_N, _N), (_C, _C), (_O, _O), (_CB, _CB),
    (_CA, _N), (_CA, _C), (_CA, _O), (_CA, _CB),
    (_N, _C), (_N, _O), (_N, _CB), (_CB, _C), (_CB, _O), (_O, _C),
    (_N, _CA), (_C, _CA), (_O, _CA), (_CB, _CA),
    (_C, _N), (_O, _N), (_CB, _N), (_C, _CB), (_O, _CB), (_C, _O),
)


def _stage1_body(x_ref, cat_ref, meta_ref, eidx_ref, coords_ref):
    x = x_ref[...]
    n = x[:, 0:3]
    ca = x[:, 3:6]
    cc = x[:, 6:9]
    b = ca - n
    c = cc - ca
    ax = b[:, 1:2] * c[:, 2:3] - b[:, 2:3] * c[:, 1:2]
    ay = b[:, 2:3] * c[:, 0:1] - b[:, 0:1] * c[:, 2:3]
    az = b[:, 0:1] * c[:, 1:2] - b[:, 1:2] * c[:, 0:1]
    a = jnp.concatenate([ax, ay, az], axis=1)
    cb = -0.58273431 * a + 0.56802827 * b - 0.54067466 * c + ca
    coords_ref[:, 0:12] = x
    coords_ref[:, 12:15] = cb
    coords_ref[:, 15:16] = meta_ref[...]

    rb = x_ref.shape[0]
    l = cat_ref.shape[1]
    dx = ca[:, 0:1] - cat_ref[0:1, :]
    dy = ca[:, 1:2] - cat_ref[1:2, :]
    dz = ca[:, 2:3] - cat_ref[2:3, :]
    d = jnp.sqrt(dx * dx + dy * dy + dz * dz + 1e-6)
    iota = lax.broadcasted_iota(jnp.int32, (rb, l), 1)
    for t in range(_K):
        m = jnp.min(d, axis=1, keepdims=True)
        idx = jnp.min(jnp.where(d == m, iota, l), axis=1, keepdims=True)
        eidx_ref[:, t:t + 1] = idx
        d = jnp.where(iota == idx, jnp.inf, d)


def _stage1(x2, cat, meta, interpret=False):
    l = x2.shape[0]
    rb = 128
    grid = (l // rb,)
    return pl.pallas_call(
        _stage1_body,
        grid=grid,
        in_specs=[
            pl.BlockSpec((rb, 12), lambda i: (i, 0)),
            pl.BlockSpec((3, l), lambda i: (0, 0)),
            pl.BlockSpec((rb, 1), lambda i: (i, 0)),
        ],
        out_specs=[
            pl.BlockSpec((rb, _K), lambda i: (i, 0)),
            pl.BlockSpec((rb, 16), lambda i: (i, 0)),
        ],
        out_shape=[
            jax.ShapeDtypeStruct((l, _K), jnp.int32),
            jax.ShapeDtypeStruct((l, 16), jnp.float32),
        ],
        interpret=interpret,
    )(x2, cat, meta)


def _stage2_gather(coords16, eidx_flat):
    ne = eidx_flat.shape[0]
    nw = 32
    bpw = ne // nw
    mesh = plsc.VectorSubcoreMesh(core_axis_name="c", subcore_axis_name="s")

    @functools.partial(
        pl.kernel,
        mesh=mesh,
        out_type=jax.ShapeDtypeStruct((ne, 16), jnp.float32),
        compiler_params=pltpu.CompilerParams(use_tc_tiling_on_sc=False),
        scratch_types=[
            pltpu.VMEM((bpw,), jnp.int32),
            pltpu.VMEM((bpw, 16), jnp.float32),
            pltpu.SemaphoreType.DMA,
        ],
    )
    def k(table_hbm, idx_hbm, out_hbm, idx_v, rows_v, sem):
        wid = lax.axis_index("s") * 2 + lax.axis_index("c")
        base = wid * bpw
        pltpu.sync_copy(idx_hbm.at[pl.ds(base, bpw)], idx_v)
        pltpu.async_copy(table_hbm.at[idx_v], rows_v, sem).wait()
        pltpu.sync_copy(rows_v, out_hbm.at[pl.ds(base, bpw)])

    return k(coords16, eidx_flat)


def _stage3_body(yt_ref, at_ref, mem_ref, mus_ref, w_ref,
                 bias_ref, g_ref, b_ref, nodew_ref, ng_ref, nb_ref,
                 e_ref, v_ref, f_ref):
    yt = yt_ref[...]
    at = at_ref[...]

    m_a = at[15:16, :]
    m_b = yt[15:16, :]
    r_a = jnp.floor(m_a * 0.25)
    r_b = jnp.floor(m_b * 0.25)
    ch_eq = (m_a - 4.0 * r_a) == (m_b - 4.0 * r_b)
    off = r_a - r_b
    dpos = jnp.where(ch_eq, jnp.clip(off + _MAXREL, 0.0, 2.0 * _MAXREL),
                     2.0 * _MAXREL + 1.0)
    io66 = lax.broadcasted_iota(
        jnp.int32, (2 * _MAXREL + 2, 1), 0).astype(jnp.float32)
    onehot = (dpos == io66).astype(jnp.float32)
    dn = (((0,), (0,)), ((), ()))
    f_ref[0:2 * _MAXREL + 2, :] = onehot

    musc = mus_ref[...]
    inv_sigma = _NUM_RBF / (22.0 - 2.0)
    for p, (ao, bo) in enumerate(_PAIRS):
        dx = at[ao:ao + 1, :] - yt[bo:bo + 1, :]
        dy = at[ao + 1:ao + 2, :] - yt[bo + 1:bo + 2, :]
        dz = at[ao + 2:ao + 3, :] - yt[bo + 2:bo + 3, :]
        dist = jnp.sqrt(dx * dx + dy * dy + dz * dz + 1e-6)
        z = (dist - musc) * inv_sigma
        base = 2 * _MAXREL + 2 + _NUM_RBF * p
        f_ref[base:base + _NUM_RBF, :] = jnp.exp(-(z * z))

    acc = lax.dot_general(f_ref[...], w_ref[...], dn,
                          preferred_element_type=jnp.float32) + bias_ref[...]
    mu = jnp.mean(acc, axis=-1, keepdims=True)
    xc = acc - mu
    var = jnp.mean(xc * xc, axis=-1, keepdims=True)
    e_ref[...] = xc / jnp.sqrt(var + 1e-5) * g_ref[...] + b_ref[...]

    mem = mem_ref[...]
    io3 = lax.broadcasted_iota(jnp.int32, (mem.shape[0], 3), 1)
    oh3 = (mem == io3).astype(jnp.float32)
    v = jnp.dot(oh3, nodew_ref[...], preferred_element_type=jnp.float32)
    vmu = jnp.mean(v, axis=-1, keepdims=True)
    vc = v - vmu
    vvar = jnp.mean(vc * vc, axis=-1, keepdims=True)
    v_ref[...] = vc / jnp.sqrt(vvar + 1e-5) * ng_ref[...] + nb_ref[...]


def _stage3(yt, at, mem2, mus_col, w466, bias_c, ln_e_g, ln_e_b,
            node_w, ln_n_g, ln_n_b, interpret=False):
    ne = yt.shape[1]
    l = mem2.shape[0]
    rb = 128
    eb = rb * _K
    nf = 2 * _MAXREL + 2 + _NUM_RBF * len(_PAIRS)
    grid = (l // rb,)
    full = lambda shape: pl.BlockSpec(shape, lambda i: tuple(0 for _ in shape))
    return pl.pallas_call(
        _stage3_body,
        grid=grid,
        in_specs=[
            pl.BlockSpec((16, eb), lambda i: (0, i)),
            pl.BlockSpec((16, eb), lambda i: (0, i)),
            pl.BlockSpec((rb, 1), lambda i: (i, 0)),
            full((_NUM_RBF, 1)),
            full((nf, 128)),
            full((1, 128)),
            full((1, 128)),
            full((1, 128)),
            full((3, 128)),
            full((1, 128)),
            full((1, 128)),
        ],
        out_specs=[
            pl.BlockSpec((eb, 128), lambda i: (i, 0)),
            pl.BlockSpec((rb, 128), lambda i: (i, 0)),
        ],
        out_shape=[
            jax.ShapeDtypeStruct((ne, 128), jnp.float32),
            jax.ShapeDtypeStruct((l, 128), jnp.float32),
        ],
        scratch_shapes=[pltpu.VMEM((nf, eb), jnp.float32)],
        interpret=interpret,
    )(yt, at, mem2, mus_col, w466, bias_c, ln_e_g, ln_e_b,
      node_w, ln_n_g, ln_n_b)


def kernel(X, mask, R_idx, chain_labels, membrane_per_residue_labels,
           pos_W, pos_b, edge_W, ln_e_g, ln_e_b, node_W, ln_n_g, ln_n_b):
    b, l = X.shape[0], X.shape[1]
    x2 = X.reshape(l, 12)
    cat = x2[:, 3:6].T
    meta = (4 * R_idx[0] + chain_labels[0]).astype(jnp.float32).reshape(l, 1)

    e_idx, coords16 = _stage1(x2, cat, meta)

    y = _stage2_gather(coords16, e_idx.reshape(l * _K))
    yt = y.T
    at = jnp.repeat(coords16.T, _K, axis=1)

    w466 = jnp.concatenate([pos_W @ edge_W[:16], edge_W[16:]], axis=0)
    bias_c = (pos_b @ edge_W[:16]).reshape(1, 128)
    mus_col = jnp.linspace(2.0, 22.0, _NUM_RBF).astype(jnp.float32).reshape(
        _NUM_RBF, 1)
    mem2 = membrane_per_residue_labels[0].astype(jnp.int32).reshape(l, 1)

    e, v = _stage3(yt, at, mem2, mus_col, w466, bias_c,
                   ln_e_g.reshape(1, 128), ln_e_b.reshape(1, 128),
                   node_W, ln_n_g.reshape(1, 128), ln_n_b.reshape(1, 128))

    return (v.reshape(b, l, 128), e.reshape(b, l, _K, 128),
            e_idx.reshape(b, l, _K))


def _run_cpu_stages(d, interpret=True):
    X, R_idx, chain = d["X"], d["R_idx"], d["chain_labels"]
    l = X.shape[1]
    x2 = X.reshape(l, 12)
    cat = x2[:, 3:6].T
    meta = (4 * R_idx[0] + chain[0]).astype(jnp.float32).reshape(l, 1)
    e_idx, coords16 = _stage1(x2, cat, meta, interpret=interpret)
    y = jnp.take_along_axis(coords16, e_idx.reshape(l * _K, 1), axis=0)
    yt = y.T
    at = jnp.repeat(coords16.T, _K, axis=1)
    w466 = jnp.concatenate([d["pos_W"] @ d["edge_W"][:16], d["edge_W"][16:]],
                           axis=0)
    bias_c = (d["pos_b"] @ d["edge_W"][:16]).reshape(1, 128)
    mus_col = jnp.linspace(2.0, 22.0, _NUM_RBF).astype(jnp.float32).reshape(
        _NUM_RBF, 1)
    mem2 = d["membrane_per_residue_labels"][0].astype(jnp.int32).reshape(l, 1)
    e, v = _stage3(yt, at, mem2, mus_col, w466, bias_c,
                   d["ln_e_g"].reshape(1, 128), d["ln_e_b"].reshape(1, 128),
                   d["node_W"], d["ln_n_g"].reshape(1, 128),
                   d["ln_n_b"].reshape(1, 128), interpret=interpret)
    return e_idx, e.reshape(l, _K, 128), v


# speedup vs baseline: 4.0426x; 1.0707x over previous
import functools

import jax
import jax.numpy as jnp
from jax import lax
from jax.experimental import pallas as pl
from jax.experimental.pallas import tpu as pltpu
from jax.experimental.pallas import tpu_sc as plsc

_K = 48
_NUM_RBF = 16
_MAXREL = 32

_N, _CA, _C, _O, _CB = 0, 3, 6, 9, 12
_PAIRS = (
    (_CA, _CA),
    (_N, _N), (_C, _C), (_O, _O), (_CB, _CB),
    (_CA, _N), (_CA, _C), (_CA, _O), (_CA, _CB),
    (_N, _C), (_N, _O), (_N, _CB), (_CB, _C), (_CB, _O), (_O, _C),
    (_N, _CA), (_C, _CA), (_O, _CA), (_CB, _CA),
    (_C, _N), (_O, _N), (_CB, _N), (_C, _CB), (_O, _CB), (_C, _O),
)


def _stage1_body(x_ref, cat_ref, meta_ref, eidx_ref, coords_ref):
    x = x_ref[...]
    n = x[:, 0:3]
    ca = x[:, 3:6]
    cc = x[:, 6:9]
    b = ca - n
    c = cc - ca
    ax = b[:, 1:2] * c[:, 2:3] - b[:, 2:3] * c[:, 1:2]
    ay = b[:, 2:3] * c[:, 0:1] - b[:, 0:1] * c[:, 2:3]
    az = b[:, 0:1] * c[:, 1:2] - b[:, 1:2] * c[:, 0:1]
    a = jnp.concatenate([ax, ay, az], axis=1)
    cb = -0.58273431 * a + 0.56802827 * b - 0.54067466 * c + ca
    coords_ref[:, 0:12] = x
    coords_ref[:, 12:15] = cb
    coords_ref[:, 15:16] = meta_ref[...]

    rb = x_ref.shape[0]
    l = cat_ref.shape[1]
    dx = ca[:, 0:1] - cat_ref[0:1, :]
    dy = ca[:, 1:2] - cat_ref[1:2, :]
    dz = ca[:, 2:3] - cat_ref[2:3, :]
    d = jnp.sqrt(dx * dx + dy * dy + dz * dz + 1e-6)
    iota = lax.broadcasted_iota(jnp.int32, (rb, l), 1)
    for t in range(_K):
        m = jnp.min(d, axis=1, keepdims=True)
        idx = jnp.min(jnp.where(d == m, iota, l), axis=1, keepdims=True)
        eidx_ref[:, t:t + 1] = idx
        d = jnp.where(iota == idx, jnp.inf, d)


def _stage1(x2, cat, meta, interpret=False):
    l = x2.shape[0]
    rb = 128
    grid = (l // rb,)
    return pl.pallas_call(
        _stage1_body,
        grid=grid,
        in_specs=[
            pl.BlockSpec((rb, 12), lambda i: (i, 0)),
            pl.BlockSpec((3, l), lambda i: (0, 0)),
            pl.BlockSpec((rb, 1), lambda i: (i, 0)),
        ],
        out_specs=[
            pl.BlockSpec((rb, _K), lambda i: (i, 0)),
            pl.BlockSpec((rb, 16), lambda i: (i, 0)),
        ],
        out_shape=[
            jax.ShapeDtypeStruct((l, _K), jnp.int32),
            jax.ShapeDtypeStruct((l, 16), jnp.float32),
        ],
        interpret=interpret,
    )(x2, cat, meta)


def _stage2_gather(coords16, eidx_flat):
    ne = eidx_flat.shape[0]
    lv = coords16.shape[0]
    nw = 32
    bpw = ne // nw
    mesh = plsc.VectorSubcoreMesh(core_axis_name="c", subcore_axis_name="s")

    @functools.partial(
        pl.kernel,
        mesh=mesh,
        out_type=jax.ShapeDtypeStruct((16 * ne,), jnp.float32),
        compiler_params=pltpu.CompilerParams(needs_layout_passes=False),
        scratch_types=[
            pltpu.VMEM((lv * 16,), jnp.float32),
            pltpu.VMEM((bpw,), jnp.int32),
            pltpu.VMEM((16 * bpw,), jnp.float32),
            pltpu.SemaphoreType.DMA,
        ],
    )
    def k(table_hbm, idx_hbm, out_hbm, table_v, idx_v, out_v, sem):
        wid = lax.axis_index("s") * 2 + lax.axis_index("c")
        base = wid * bpw
        pltpu.sync_copy(table_hbm, table_v)
        pltpu.sync_copy(idx_hbm.at[pl.ds(base, bpw)], idx_v)

        def body(g, carry):
            iv = idx_v[pl.ds(g * 16, 16)] * 16
            for c in range(16):
                out_v[pl.ds(c * bpw + g * 16, 16)] = plsc.load_gather(
                    table_v, [iv + c])
            return carry

        lax.fori_loop(0, bpw // 16, body, 0)
        for c in range(16):
            pltpu.sync_copy(out_v.at[pl.ds(c * bpw, bpw)],
                            out_hbm.at[pl.ds(c * ne + base, bpw)])

    return k(coords16.reshape(lv * 16), eidx_flat)


def _stage3_body(yt_ref, ct_ref, rep_ref, mem_ref, mus_ref, w_ref,
                 bias_ref, g_ref, b_ref, nodew_ref, ng_ref, nb_ref,
                 e_ref, v_ref, f_ref):
    yt = yt_ref[...]
    at = lax.dot_general(ct_ref[...], rep_ref[...], (((1,), (0,)), ((), ())),
                         precision=lax.Precision.HIGHEST,
                         preferred_element_type=jnp.float32)

    m_a = at[15:16, :]
    m_b = yt[15:16, :]
    r_a = jnp.floor(m_a * 0.25)
    r_b = jnp.floor(m_b * 0.25)
    ch_eq = (m_a - 4.0 * r_a) == (m_b - 4.0 * r_b)
    off = r_a - r_b
    dpos = jnp.where(ch_eq, jnp.clip(off + _MAXREL, 0.0, 2.0 * _MAXREL),
                     2.0 * _MAXREL + 1.0)
    io66 = lax.broadcasted_iota(
        jnp.int32, (2 * _MAXREL + 2, 1), 0).astype(jnp.float32)
    onehot = (dpos == io66).astype(jnp.float32)
    dn = (((0,), (0,)), ((), ()))
    f_ref[0:2 * _MAXREL + 2, :] = onehot

    musc = mus_ref[...]
    inv_sigma = _NUM_RBF / (22.0 - 2.0)
    for p, (ao, bo) in enumerate(_PAIRS):
        dx = at[ao:ao + 1, :] - yt[bo:bo + 1, :]
        dy = at[ao + 1:ao + 2, :] - yt[bo + 1:bo + 2, :]
        dz = at[ao + 2:ao + 3, :] - yt[bo + 2:bo + 3, :]
        dist = jnp.sqrt(dx * dx + dy * dy + dz * dz + 1e-6)
        z = (dist - musc) * inv_sigma
        base = 2 * _MAXREL + 2 + _NUM_RBF * p
        f_ref[base:base + _NUM_RBF, :] = jnp.exp(-(z * z))

    acc = lax.dot_general(f_ref[...], w_ref[...], dn,
                          preferred_element_type=jnp.float32) + bias_ref[...]
    mu = jnp.mean(acc, axis=-1, keepdims=True)
    xc = acc - mu
    var = jnp.mean(xc * xc, axis=-1, keepdims=True)
    e_ref[...] = xc / jnp.sqrt(var + 1e-5) * g_ref[...] + b_ref[...]

    mem = mem_ref[...]
    io3 = lax.broadcasted_iota(jnp.int32, (mem.shape[0], 3), 1)
    oh3 = (mem == io3).astype(jnp.float32)
    v = jnp.dot(oh3, nodew_ref[...], preferred_element_type=jnp.float32)
    vmu = jnp.mean(v, axis=-1, keepdims=True)
    vc = v - vmu
    vvar = jnp.mean(vc * vc, axis=-1, keepdims=True)
    v_ref[...] = vc / jnp.sqrt(vvar + 1e-5) * ng_ref[...] + nb_ref[...]


def _stage3(yt, coordst, rep, mem2, mus_col, w466, bias_c, ln_e_g, ln_e_b,
            node_w, ln_n_g, ln_n_b, interpret=False):
    ne = yt.shape[1]
    l = mem2.shape[0]
    rb = 128
    eb = rb * _K
    nf = 2 * _MAXREL + 2 + _NUM_RBF * len(_PAIRS)
    grid = (l // rb,)
    full = lambda shape: pl.BlockSpec(shape, lambda i: tuple(0 for _ in shape))
    return pl.pallas_call(
        _stage3_body,
        grid=grid,
        in_specs=[
            pl.BlockSpec((16, eb), lambda i: (0, i)),
            pl.BlockSpec((16, rb), lambda i: (0, i)),
            full((rb, eb)),
            pl.BlockSpec((rb, 1), lambda i: (i, 0)),
            full((_NUM_RBF, 1)),
            full((nf, 128)),
            full((1, 128)),
            full((1, 128)),
            full((1, 128)),
            full((3, 128)),
            full((1, 128)),
            full((1, 128)),
        ],
        out_specs=[
            pl.BlockSpec((eb, 128), lambda i: (i, 0)),
            pl.BlockSpec((rb, 128), lambda i: (i, 0)),
        ],
        out_shape=[
            jax.ShapeDtypeStruct((ne, 128), jnp.float32),
            jax.ShapeDtypeStruct((l, 128), jnp.float32),
        ],
        scratch_shapes=[pltpu.VMEM((nf, eb), jnp.float32)],
        interpret=interpret,
    )(yt, coordst, rep, mem2, mus_col, w466, bias_c, ln_e_g, ln_e_b,
      node_w, ln_n_g, ln_n_b)


def kernel(X, mask, R_idx, chain_labels, membrane_per_residue_labels,
           pos_W, pos_b, edge_W, ln_e_g, ln_e_b, node_W, ln_n_g, ln_n_b):
    b, l = X.shape[0], X.shape[1]
    x2 = X.reshape(l, 12)
    cat = x2[:, 3:6].T
    meta = (4 * R_idx[0] + chain_labels[0]).astype(jnp.float32).reshape(l, 1)

    e_idx, coords16 = _stage1(x2, cat, meta)

    yt = _stage2_gather(coords16, e_idx.reshape(l * _K)).reshape(16, l * _K)
    rep = (lax.broadcasted_iota(jnp.int32, (128, 128 * _K), 1) // _K ==
           lax.broadcasted_iota(jnp.int32, (128, 128 * _K), 0)
           ).astype(jnp.float32)

    w466 = jnp.concatenate([pos_W @ edge_W[:16], edge_W[16:]], axis=0)
    bias_c = (pos_b @ edge_W[:16]).reshape(1, 128)
    mus_col = jnp.linspace(2.0, 22.0, _NUM_RBF).astype(jnp.float32).reshape(
        _NUM_RBF, 1)
    mem2 = membrane_per_residue_labels[0].astype(jnp.int32).reshape(l, 1)

    e, v = _stage3(yt, coords16.T, rep, mem2, mus_col, w466, bias_c,
                   ln_e_g.reshape(1, 128), ln_e_b.reshape(1, 128),
                   node_W, ln_n_g.reshape(1, 128), ln_n_b.reshape(1, 128))

    return (v.reshape(b, l, 128), e.reshape(b, l, _K, 128),
            e_idx.reshape(b, l, _K))


def _run_cpu_stages(d, interpret=True):
    X, R_idx, chain = d["X"], d["R_idx"], d["chain_labels"]
    l = X.shape[1]
    x2 = X.reshape(l, 12)
    cat = x2[:, 3:6].T
    meta = (4 * R_idx[0] + chain[0]).astype(jnp.float32).reshape(l, 1)
    e_idx, coords16 = _stage1(x2, cat, meta, interpret=interpret)
    y = jnp.take_along_axis(coords16, e_idx.reshape(l * _K, 1), axis=0)
    yt = y.T
    rep = (lax.broadcasted_iota(jnp.int32, (128, 128 * _K), 1) // _K ==
           lax.broadcasted_iota(jnp.int32, (128, 128 * _K), 0)
           ).astype(jnp.float32)
    w466 = jnp.concatenate([d["pos_W"] @ d["edge_W"][:16], d["edge_W"][16:]],
                           axis=0)
    bias_c = (d["pos_b"] @ d["edge_W"][:16]).reshape(1, 128)
    mus_col = jnp.linspace(2.0, 22.0, _NUM_RBF).astype(jnp.float32).reshape(
        _NUM_RBF, 1)
    mem2 = d["membrane_per_residue_labels"][0].astype(jnp.int32).reshape(l, 1)
    e, v = _stage3(yt, coords16.T, rep, mem2, mus_col, w466, bias_c,
                   d["ln_e_g"].reshape(1, 128), d["ln_e_b"].reshape(1, 128),
                   d["node_W"], d["ln_n_g"].reshape(1, 128),
                   d["ln_n_b"].reshape(1, 128), interpret=interpret)
    return e_idx, e.reshape(l, _K, 128), v


# speedup vs baseline: 4.0871x; 1.0110x over previous
import functools

import jax
import jax.numpy as jnp
from jax import lax
from jax.experimental import pallas as pl
from jax.experimental.pallas import tpu as pltpu
from jax.experimental.pallas import tpu_sc as plsc

_K = 48
_NUM_RBF = 16
_MAXREL = 32

_N, _CA, _C, _O, _CB = 0, 3, 6, 9, 12
_PAIRS = (
    (_CA, _CA),
    (_N, _N), (_C, _C), (_O, _O), (_CB, _CB),
    (_CA, _N), (_CA, _C), (_CA, _O), (_CA, _CB),
    (_N, _C), (_N, _O), (_N, _CB), (_CB, _C), (_CB, _O), (_O, _C),
    (_N, _CA), (_C, _CA), (_O, _CA), (_CB, _CA),
    (_C, _N), (_O, _N), (_CB, _N), (_C, _CB), (_O, _CB), (_C, _O),
)


def _stage1_body(x_ref, cat_ref, meta_ref, eidx_ref, coords_ref):
    x = x_ref[...]
    n = x[:, 0:3]
    ca = x[:, 3:6]
    cc = x[:, 6:9]
    b = ca - n
    c = cc - ca
    ax = b[:, 1:2] * c[:, 2:3] - b[:, 2:3] * c[:, 1:2]
    ay = b[:, 2:3] * c[:, 0:1] - b[:, 0:1] * c[:, 2:3]
    az = b[:, 0:1] * c[:, 1:2] - b[:, 1:2] * c[:, 0:1]
    a = jnp.concatenate([ax, ay, az], axis=1)
    cb = -0.58273431 * a + 0.56802827 * b - 0.54067466 * c + ca
    coords_ref[:, 0:12] = x
    coords_ref[:, 12:15] = cb
    coords_ref[:, 15:16] = meta_ref[...]

    rb = x_ref.shape[0]
    l = cat_ref.shape[1]
    dx = ca[:, 0:1] - cat_ref[0:1, :]
    dy = ca[:, 1:2] - cat_ref[1:2, :]
    dz = ca[:, 2:3] - cat_ref[2:3, :]
    d = jnp.sqrt(dx * dx + dy * dy + dz * dz + 1e-6)
    iota = lax.broadcasted_iota(jnp.int32, (rb, l), 1)
    for t in range(_K):
        m = jnp.min(d, axis=1, keepdims=True)
        idx = jnp.min(jnp.where(d == m, iota, l), axis=1, keepdims=True)
        eidx_ref[:, t:t + 1] = idx
        d = jnp.where(iota == idx, jnp.inf, d)


def _stage1(x2, cat, meta, interpret=False):
    l = x2.shape[0]
    rb = 128
    grid = (l // rb,)
    return pl.pallas_call(
        _stage1_body,
        grid=grid,
        in_specs=[
            pl.BlockSpec((rb, 12), lambda i: (i, 0)),
            pl.BlockSpec((3, l), lambda i: (0, 0)),
            pl.BlockSpec((rb, 1), lambda i: (i, 0)),
        ],
        out_specs=[
            pl.BlockSpec((rb, _K), lambda i: (i, 0)),
            pl.BlockSpec((rb, 16), lambda i: (i, 0)),
        ],
        out_shape=[
            jax.ShapeDtypeStruct((l, _K), jnp.int32),
            jax.ShapeDtypeStruct((l, 16), jnp.float32),
        ],
        interpret=interpret,
    )(x2, cat, meta)


def _stage2_gather(coords16, eidx_flat):
    ne = eidx_flat.shape[0]
    lv = coords16.shape[0]
    nw = 32
    bpw = ne // nw
    mesh = plsc.VectorSubcoreMesh(core_axis_name="c", subcore_axis_name="s")

    @functools.partial(
        pl.kernel,
        mesh=mesh,
        out_type=jax.ShapeDtypeStruct((16, ne), jnp.float32),
        compiler_params=pltpu.CompilerParams(needs_layout_passes=False),
        scratch_types=[
            pltpu.VMEM((lv * 16,), jnp.float32),
            pltpu.VMEM((bpw,), jnp.int32),
            pltpu.VMEM((16 * bpw,), jnp.float32),
            pltpu.SemaphoreType.DMA,
        ],
    )
    def k(table_hbm, idx_hbm, out_hbm, table_v, idx_v, out_v, sem):
        wid = lax.axis_index("s") * 2 + lax.axis_index("c")
        base = wid * bpw
        pltpu.sync_copy(table_hbm, table_v)
        pltpu.sync_copy(idx_hbm.at[pl.ds(base, bpw)], idx_v)

        def body(g, carry):
            iv = idx_v[pl.ds(g * 16, 16)] * 16
            for c in range(16):
                out_v[pl.ds(c * bpw + g * 16, 16)] = plsc.load_gather(
                    table_v, [iv + c])
            return carry

        lax.fori_loop(0, bpw // 16, body, 0)
        for c in range(16):
            pltpu.sync_copy(out_v.at[pl.ds(c * bpw, bpw)],
                            out_hbm.at[c, pl.ds(base, bpw)])

    return k(coords16.reshape(lv * 16), eidx_flat)


def _stage3_body(yt_ref, ct_ref, rep_ref, mem_ref, mus_ref, w_ref,
                 bias_ref, g_ref, b_ref, nodew_ref, ng_ref, nb_ref,
                 e_ref, v_ref, f_ref):
    yt = yt_ref[...]
    at = lax.dot_general(ct_ref[...], rep_ref[...], (((1,), (0,)), ((), ())),
                         precision=lax.Precision.HIGHEST,
                         preferred_element_type=jnp.float32)

    m_a = at[15:16, :]
    m_b = yt[15:16, :]
    r_a = jnp.floor(m_a * 0.25)
    r_b = jnp.floor(m_b * 0.25)
    ch_eq = (m_a - 4.0 * r_a) == (m_b - 4.0 * r_b)
    off = r_a - r_b
    dpos = jnp.where(ch_eq, jnp.clip(off + _MAXREL, 0.0, 2.0 * _MAXREL),
                     2.0 * _MAXREL + 1.0)
    io66 = lax.broadcasted_iota(
        jnp.int32, (2 * _MAXREL + 2, 1), 0).astype(jnp.float32)
    onehot = (dpos == io66).astype(jnp.float32)
    dn = (((0,), (0,)), ((), ()))
    f_ref[0:2 * _MAXREL + 2, :] = onehot

    musc = mus_ref[...]
    inv_sigma = _NUM_RBF / (22.0 - 2.0)
    for p, (ao, bo) in enumerate(_PAIRS):
        dx = at[ao:ao + 1, :] - yt[bo:bo + 1, :]
        dy = at[ao + 1:ao + 2, :] - yt[bo + 1:bo + 2, :]
        dz = at[ao + 2:ao + 3, :] - yt[bo + 2:bo + 3, :]
        dist = jnp.sqrt(dx * dx + dy * dy + dz * dz + 1e-6)
        z = (dist - musc) * inv_sigma
        base = 2 * _MAXREL + 2 + _NUM_RBF * p
        f_ref[base:base + _NUM_RBF, :] = jnp.exp(-(z * z))

    acc = lax.dot_general(f_ref[...], w_ref[...], dn,
                          preferred_element_type=jnp.float32) + bias_ref[...]
    mu = jnp.mean(acc, axis=-1, keepdims=True)
    xc = acc - mu
    var = jnp.mean(xc * xc, axis=-1, keepdims=True)
    e_ref[...] = xc / jnp.sqrt(var + 1e-5) * g_ref[...] + b_ref[...]

    mem = mem_ref[...]
    io3 = lax.broadcasted_iota(jnp.int32, (mem.shape[0], 3), 1)
    oh3 = (mem == io3).astype(jnp.float32)
    v = jnp.dot(oh3, nodew_ref[...], preferred_element_type=jnp.float32)
    vmu = jnp.mean(v, axis=-1, keepdims=True)
    vc = v - vmu
    vvar = jnp.mean(vc * vc, axis=-1, keepdims=True)
    v_ref[...] = vc / jnp.sqrt(vvar + 1e-5) * ng_ref[...] + nb_ref[...]


def _stage3(yt, coordst, rep, mem2, mus_col, w466, bias_c, ln_e_g, ln_e_b,
            node_w, ln_n_g, ln_n_b, interpret=False):
    ne = yt.shape[1]
    l = mem2.shape[0]
    rb = 128
    eb = rb * _K
    nf = 2 * _MAXREL + 2 + _NUM_RBF * len(_PAIRS)
    grid = (l // rb,)
    full = lambda shape: pl.BlockSpec(shape, lambda i: tuple(0 for _ in shape))
    return pl.pallas_call(
        _stage3_body,
        grid=grid,
        in_specs=[
            pl.BlockSpec((16, eb), lambda i: (0, i)),
            pl.BlockSpec((16, rb), lambda i: (0, i)),
            full((rb, eb)),
            pl.BlockSpec((rb, 1), lambda i: (i, 0)),
            full((_NUM_RBF, 1)),
            full((nf, 128)),
            full((1, 128)),
            full((1, 128)),
            full((1, 128)),
            full((3, 128)),
            full((1, 128)),
            full((1, 128)),
        ],
        out_specs=[
            pl.BlockSpec((eb, 128), lambda i: (i, 0)),
            pl.BlockSpec((rb, 128), lambda i: (i, 0)),
        ],
        out_shape=[
            jax.ShapeDtypeStruct((ne, 128), jnp.float32),
            jax.ShapeDtypeStruct((l, 128), jnp.float32),
        ],
        scratch_shapes=[pltpu.VMEM((nf, eb), jnp.float32)],
        interpret=interpret,
    )(yt, coordst, rep, mem2, mus_col, w466, bias_c, ln_e_g, ln_e_b,
      node_w, ln_n_g, ln_n_b)


def kernel(X, mask, R_idx, chain_labels, membrane_per_residue_labels,
           pos_W, pos_b, edge_W, ln_e_g, ln_e_b, node_W, ln_n_g, ln_n_b):
    b, l = X.shape[0], X.shape[1]
    x2 = X.reshape(l, 12)
    cat = x2[:, 3:6].T
    meta = (4 * R_idx[0] + chain_labels[0]).astype(jnp.float32).reshape(l, 1)

    e_idx, coords16 = _stage1(x2, cat, meta)

    yt = _stage2_gather(coords16, e_idx.reshape(l * _K))
    rep = (lax.broadcasted_iota(jnp.int32, (128, 128 * _K), 1) // _K ==
           lax.broadcasted_iota(jnp.int32, (128, 128 * _K), 0)
           ).astype(jnp.float32)

    w466 = jnp.concatenate([pos_W @ edge_W[:16], edge_W[16:]], axis=0)
    bias_c = (pos_b @ edge_W[:16]).reshape(1, 128)
    mus_col = jnp.linspace(2.0, 22.0, _NUM_RBF).astype(jnp.float32).reshape(
        _NUM_RBF, 1)
    mem2 = membrane_per_residue_labels[0].astype(jnp.int32).reshape(l, 1)

    e, v = _stage3(yt, coords16.T, rep, mem2, mus_col, w466, bias_c,
                   ln_e_g.reshape(1, 128), ln_e_b.reshape(1, 128),
                   node_W, ln_n_g.reshape(1, 128), ln_n_b.reshape(1, 128))

    return (v.reshape(b, l, 128), e.reshape(b, l, _K, 128),
            e_idx.reshape(b, l, _K))


def _run_cpu_stages(d, interpret=True):
    X, R_idx, chain = d["X"], d["R_idx"], d["chain_labels"]
    l = X.shape[1]
    x2 = X.reshape(l, 12)
    cat = x2[:, 3:6].T
    meta = (4 * R_idx[0] + chain[0]).astype(jnp.float32).reshape(l, 1)
    e_idx, coords16 = _stage1(x2, cat, meta, interpret=interpret)
    y = jnp.take_along_axis(coords16, e_idx.reshape(l * _K, 1), axis=0)
    yt = y.T
    rep = (lax.broadcasted_iota(jnp.int32, (128, 128 * _K), 1) // _K ==
           lax.broadcasted_iota(jnp.int32, (128, 128 * _K), 0)
           ).astype(jnp.float32)
    w466 = jnp.concatenate([d["pos_W"] @ d["edge_W"][:16], d["edge_W"][16:]],
                           axis=0)
    bias_c = (d["pos_b"] @ d["edge_W"][:16]).reshape(1, 128)
    mus_col = jnp.linspace(2.0, 22.0, _NUM_RBF).astype(jnp.float32).reshape(
        _NUM_RBF, 1)
    mem2 = d["membrane_per_residue_labels"][0].astype(jnp.int32).reshape(l, 1)
    e, v = _stage3(yt, coords16.T, rep, mem2, mus_col, w466, bias_c,
                   d["ln_e_g"].reshape(1, 128), d["ln_e_b"].reshape(1, 128),
                   d["node_W"], d["ln_n_g"].reshape(1, 128),
                   d["ln_n_b"].reshape(1, 128), interpret=interpret)
    return e_idx, e.reshape(l, _K, 128), v
